# baseline (speedup 1.0000x reference)
import functools

import jax
import jax.numpy as jnp
from jax import lax
from jax.experimental import pallas as pl
from jax.experimental.pallas import tpu as pltpu
from jax.experimental.pallas import tpu_sc as plsc

EMBED_DIM = 16
HIDDEN_DIM = 32
VOCAB = 21
VPAD = 24
CPR = 4 * VPAD
SEG_START = (0, 11, 22, 56)
SEG_LEN = (11, 11, 34, 11)
TOK_TOTAL = 67
NW = 32
NB = 4096


def _sc_body(toks_hbm, cnt_hbm, toks_v, cnt_v):
    cid = lax.axis_index("c")
    sid = lax.axis_index("s")
    wid = cid * 16 + sid
    rpw = toks_v.shape[1]
    pltpu.sync_copy(toks_hbm.at[:, pl.ds(wid * rpw, rpw)], toks_v)

    zeros16 = jnp.zeros((16,), jnp.float32)
    zunroll = 16

    def zbody(i, carry):
        for u in range(zunroll):
            cnt_v[pl.ds((i * zunroll + u) * 16, 16)] = zeros16
        return carry

    lax.fori_loop(0, rpw * CPR // 16 // zunroll, zbody, 0)

    laneiota = lax.iota(jnp.int32, 16)
    laneoff = laneiota * CPR
    ones16 = jnp.ones((16,), jnp.float32)

    def gbody(g, carry):
        base = laneoff + g * (16 * CPR)
        for s in range(4):
            bs = base + s * VPAD
            for p in range(SEG_LEN[s]):
                tok = toks_v[SEG_START[s] + p, pl.ds(g * 16, 16)]
                plsc.addupdate_scatter(cnt_v, [tok + bs], ones16)
        return carry

    lax.fori_loop(0, rpw // 16, gbody, 0)
    pltpu.sync_copy(cnt_v, cnt_hbm.at[wid])


@functools.partial(jax.jit, static_argnames=("sc_rows",))
def _sc_counts(toks_t, sc_rows):
    rpw = sc_rows // NW
    mesh = plsc.VectorSubcoreMesh(core_axis_name="c", subcore_axis_name="s")
    f = pl.kernel(
        _sc_body,
        out_type=jax.ShapeDtypeStruct((NW, rpw * CPR), jnp.float32),
        mesh=mesh,
        scratch_types=[
            pltpu.VMEM((TOK_TOTAL, rpw), jnp.int32),
            pltpu.VMEM((rpw * CPR,), jnp.float32),
        ],
        compiler_params=pltpu.CompilerParams(needs_layout_passes=False),
    )
    return f(toks_t)


NB_TC = 4096


def _tc_body(toks_ref, scal_ref, bd_ref, w1t_ref, b1_ref, w2at_ref,
             w2bt_ref, b2_ref, w3t_ref, b3_ref, out_ref):
    toks = toks_ref[...]
    scal = scal_ref[...]
    viota = jax.lax.broadcasted_iota(jnp.int16, (VPAD, NB_TC), 0)
    slabs = []
    for seg_idx in range(4):
        start, length = SEG_START[seg_idx], SEG_LEN[seg_idx]
        parts = [jnp.zeros((VPAD, NB_TC), jnp.int16) for _ in range(2)]
        for p in range(length):
            tokp = toks[start + p, :][None, :]
            eq = (viota == jnp.broadcast_to(tokp, (VPAD, NB_TC)))
            parts[p % 2] = parts[p % 2] + eq.astype(jnp.int16)
        slab = (parts[0] + parts[1]).astype(jnp.float32)
        denom = jnp.maximum(float(length) - slab[0:1, :], 1.0)
        slabs.append(slab * (1.0 / denom))
    counts = jnp.concatenate(slabs, axis=0)
    pooled = jnp.dot(bd_ref[...], counts,
                     preferred_element_type=jnp.float32)
    sf = jnp.maximum(
        jnp.dot(w1t_ref[...], scal,
                preferred_element_type=jnp.float32) + b1_ref[...], 0.0)
    h = jnp.maximum(
        jnp.dot(w2at_ref[...], pooled, preferred_element_type=jnp.float32)
        + jnp.dot(w2bt_ref[...], sf, preferred_element_type=jnp.float32)
        + b2_ref[...], 0.0)
    out_ref[...] = (jnp.dot(w3t_ref[...], h,
                            preferred_element_type=jnp.float32)
                    + b3_ref[...])


@functools.partial(jax.jit, static_argnames=("col0",))
def _tc_call(toks_t, scalars_t, bd, w1t, b1c, w2at, w2bt, b2c, w3t, b3c,
             col0=0):
    batch = toks_t.shape[1] - col0
    grid = (batch // NB_TC,)
    off = col0 // NB_TC
    return pl.pallas_call(
        _tc_body,
        grid=grid,
        in_specs=[
            pl.BlockSpec((TOK_TOTAL, NB_TC), lambda j: (0, j + off)),
            pl.BlockSpec((10, NB_TC), lambda j: (0, j + off)),
            pl.BlockSpec((4 * EMBED_DIM, CPR), lambda j: (0, 0)),
            pl.BlockSpec((HIDDEN_DIM, 10), lambda j: (0, 0)),
            pl.BlockSpec((HIDDEN_DIM, 1), lambda j: (0, 0)),
            pl.BlockSpec((HIDDEN_DIM, 4 * EMBED_DIM), lambda j: (0, 0)),
            pl.BlockSpec((HIDDEN_DIM, HIDDEN_DIM), lambda j: (0, 0)),
            pl.BlockSpec((HIDDEN_DIM, 1), lambda j: (0, 0)),
            pl.BlockSpec((1, HIDDEN_DIM), lambda j: (0, 0)),
            pl.BlockSpec((1, 1), lambda j: (0, 0)),
        ],
        out_specs=pl.BlockSpec((1, NB_TC), lambda j: (0, j)),
        out_shape=jax.ShapeDtypeStruct((1, batch), jnp.float32),
        compiler_params=pltpu.CompilerParams(
            dimension_semantics=("parallel",)),
    )(toks_t, scalars_t, bd, w1t, b1c, w2at, w2bt, b2c, w3t, b3c)


def _head_body(cnt_ref, scal_ref, bdt_ref, sel0_ref, rep_ref, w1_ref,
               b1_ref, w2a_ref, w2b_ref, b2_ref, w3_ref, b3_ref, out_ref):
    counts = cnt_ref[...]
    li = lax.broadcasted_iota(jnp.int32, (1, 4), 1)
    seg_lens = jnp.where(li == 2, 34.0, 11.0)
    z = jnp.dot(counts, sel0_ref[...],
                preferred_element_type=jnp.float32)
    recip = 1.0 / jnp.maximum(seg_lens - z, 1.0)
    pooled_raw = jnp.dot(counts, bdt_ref[...],
                         preferred_element_type=jnp.float32)
    scale = jnp.dot(recip, rep_ref[...],
                    preferred_element_type=jnp.float32)
    pooled = pooled_raw * scale
    sf = jnp.maximum(
        jnp.dot(scal_ref[...], w1_ref[...],
                preferred_element_type=jnp.float32) + b1_ref[...], 0.0)
    h = jnp.maximum(
        jnp.dot(pooled, w2a_ref[...], preferred_element_type=jnp.float32)
        + jnp.dot(sf, w2b_ref[...], preferred_element_type=jnp.float32)
        + b2_ref[...], 0.0)
    out_ref[...] = (jnp.dot(h, w3_ref[...],
                            preferred_element_type=jnp.float32)
                    + b3_ref[...])


@jax.jit
def _head_call(cnt, scalars, bdt, sel0, rep, w1, b1r, w2a, w2b, b2r, w3,
               b3r):
    batch = cnt.shape[0]
    grid = (batch // NB,)
    return pl.pallas_call(
        _head_body,
        grid=grid,
        in_specs=[
            pl.BlockSpec((NB, CPR), lambda j: (j, 0)),
            pl.BlockSpec((NB, 10), lambda j: (j, 0)),
            pl.BlockSpec((CPR, 4 * EMBED_DIM), lambda j: (0, 0)),
            pl.BlockSpec((CPR, 4), lambda j: (0, 0)),
            pl.BlockSpec((4, 4 * EMBED_DIM), lambda j: (0, 0)),
            pl.BlockSpec((10, HIDDEN_DIM), lambda j: (0, 0)),
            pl.BlockSpec((1, HIDDEN_DIM), lambda j: (0, 0)),
            pl.BlockSpec((4 * EMBED_DIM, HIDDEN_DIM), lambda j: (0, 0)),
            pl.BlockSpec((HIDDEN_DIM, HIDDEN_DIM), lambda j: (0, 0)),
            pl.BlockSpec((1, HIDDEN_DIM), lambda j: (0, 0)),
            pl.BlockSpec((HIDDEN_DIM, 1), lambda j: (0, 0)),
            pl.BlockSpec((1, 1), lambda j: (0, 0)),
        ],
        out_specs=pl.BlockSpec((NB, 1), lambda j: (j, 0)),
        out_shape=jax.ShapeDtypeStruct((batch, 1), jnp.float32),
        compiler_params=pltpu.CompilerParams(
            dimension_semantics=("parallel",)),
    )(cnt, scalars, bdt, sel0, rep, w1, b1r, w2a, w2b, b2r, w3, b3r)


SC_FRAC_NUM = 1
SC_FRAC_DEN = 4


def kernel(mut_tokens, wt_tokens, hla_tokens, delta_tokens, scalars,
           embedding, W1, b1, W2, b2, W3, b3):
    batch = mut_tokens.shape[0]
    sc_rows = (batch * SC_FRAC_NUM // SC_FRAC_DEN) // (NW * 16) * (NW * 16)
    rpw = sc_rows // NW
    toks_t16 = jnp.concatenate(
        [mut_tokens.T, wt_tokens.T, hla_tokens.T, delta_tokens.T],
        axis=0).astype(jnp.int16)
    toks_t_sc = jnp.concatenate(
        [mut_tokens[:sc_rows].T, wt_tokens[:sc_rows].T,
         hla_tokens[:sc_rows].T, delta_tokens[:sc_rows].T],
        axis=0).astype(jnp.int32)

    cnt = _sc_counts(toks_t_sc, sc_rows=sc_rows).reshape(sc_rows, CPR)

    ezt = embedding.at[0].set(0.0)
    ezt = jnp.pad(ezt, ((0, VPAD - VOCAB), (0, 0)))
    bdt = jnp.zeros((CPR, 4 * EMBED_DIM), jnp.float32)
    sel0 = jnp.zeros((CPR, 4), jnp.float32)
    rep = jnp.zeros((4, 4 * EMBED_DIM), jnp.float32)
    for s in range(4):
        bdt = bdt.at[s * VPAD:(s + 1) * VPAD,
                     s * EMBED_DIM:(s + 1) * EMBED_DIM].set(ezt)
        sel0 = sel0.at[s * VPAD, s].set(1.0)
        rep = rep.at[s, s * EMBED_DIM:(s + 1) * EMBED_DIM].set(1.0)

    out_tc = _tc_call(toks_t16, scalars.T, bdt.T,
                      W1.T, b1[:, None], W2[:64].T, W2[64:].T, b2[:, None],
                      W3.T, b3[:, None], col0=sc_rows)

    out_sc = _head_call(cnt, scalars[:sc_rows], bdt, sel0, rep, W1,
                        b1[None, :], W2[:64], W2[64:], b2[None, :], W3,
                        b3[None, :])
    return jnp.concatenate([out_sc[:, 0], out_tc[0]])


# speedup vs baseline: 1.2737x; 1.2737x over previous
import functools

import jax
import jax.numpy as jnp
from jax import lax
from jax.experimental import pallas as pl
from jax.experimental.pallas import tpu as pltpu
from jax.experimental.pallas import tpu_sc as plsc

EMBED_DIM = 16
HIDDEN_DIM = 32
VOCAB = 21
VPAD = 24
CPR = 4 * VPAD
SEG_START = (0, 11, 22, 56)
SEG_LEN = (11, 11, 34, 11)
TOK_TOTAL = 67
NW = 32
NB = 4096


def _sc_body(toks_hbm, cnt_hbm, toks_v, cnt_v):
    cid = lax.axis_index("c")
    sid = lax.axis_index("s")
    wid = cid * 16 + sid
    rpw = toks_v.shape[1]
    pltpu.sync_copy(toks_hbm.at[:, pl.ds(wid * rpw, rpw)], toks_v)

    zeros16 = jnp.zeros((16,), jnp.float32)

    def zbody(i, carry):
        for slot in range(CPR):
            cnt_v[slot, pl.ds(i * 16, 16)] = zeros16
        return carry

    lax.fori_loop(0, rpw // 16, zbody, 0)

    laneiota = lax.iota(jnp.int32, 16)
    ones16 = jnp.ones((16,), jnp.float32)

    def gbody(g, carry):
        rows = laneiota + g * 16
        for s in range(4):
            for p in range(SEG_LEN[s]):
                tok = toks_v[SEG_START[s] + p, pl.ds(g * 16, 16)]
                plsc.addupdate_scatter(cnt_v, [tok + s * VPAD, rows],
                                       ones16)
        return carry

    lax.fori_loop(0, rpw // 16, gbody, 0)
    pltpu.sync_copy(cnt_v, cnt_hbm.at[:, pl.ds(wid * rpw, rpw)])


@functools.partial(jax.jit, static_argnames=("sc_rows",))
def _sc_counts(toks_t, sc_rows):
    rpw = sc_rows // NW
    mesh = plsc.VectorSubcoreMesh(core_axis_name="c", subcore_axis_name="s")
    f = pl.kernel(
        _sc_body,
        out_type=jax.ShapeDtypeStruct((CPR, sc_rows), jnp.float32),
        mesh=mesh,
        scratch_types=[
            pltpu.VMEM((TOK_TOTAL, rpw), jnp.int32),
            pltpu.VMEM((CPR, rpw), jnp.float32),
        ],
        compiler_params=pltpu.CompilerParams(needs_layout_passes=False),
    )
    return f(toks_t)


NB_TC = 4096


def _tc_body(toks_ref, scal_ref, bd_ref, w1t_ref, b1_ref, w2at_ref,
             w2bt_ref, b2_ref, w3t_ref, b3_ref, out_ref):
    toks = toks_ref[...]
    scal = scal_ref[...]
    viota = jax.lax.broadcasted_iota(jnp.int32, (VPAD, NB_TC), 0)
    slabs = []
    for seg_idx in range(4):
        start, length = SEG_START[seg_idx], SEG_LEN[seg_idx]
        parts = [jnp.zeros((VPAD, NB_TC), jnp.float32) for _ in range(2)]
        for p in range(length):
            tokp = toks[start + p, :][None, :]
            parts[p % 2] = parts[p % 2] + jnp.where(viota == tokp, 1.0, 0.0)
        slab = parts[0] + parts[1]
        denom = jnp.maximum(float(length) - slab[0:1, :], 1.0)
        slabs.append(slab * (1.0 / denom))
    counts = jnp.concatenate(slabs, axis=0)
    pooled = jnp.dot(bd_ref[...], counts,
                     preferred_element_type=jnp.float32)
    sf = jnp.maximum(
        jnp.dot(w1t_ref[...], scal,
                preferred_element_type=jnp.float32) + b1_ref[...], 0.0)
    h = jnp.maximum(
        jnp.dot(w2at_ref[...], pooled, preferred_element_type=jnp.float32)
        + jnp.dot(w2bt_ref[...], sf, preferred_element_type=jnp.float32)
        + b2_ref[...], 0.0)
    out_ref[...] = (jnp.dot(w3t_ref[...], h,
                            preferred_element_type=jnp.float32)
                    + b3_ref[...])


@functools.partial(jax.jit, static_argnames=("col0",))
def _tc_call(toks_t, scalars_t, bd, w1t, b1c, w2at, w2bt, b2c, w3t, b3c,
             col0=0):
    batch = toks_t.shape[1] - col0
    grid = (batch // NB_TC,)
    off = col0 // NB_TC
    return pl.pallas_call(
        _tc_body,
        grid=grid,
        in_specs=[
            pl.BlockSpec((TOK_TOTAL, NB_TC), lambda j: (0, j + off)),
            pl.BlockSpec((10, NB_TC), lambda j: (0, j + off)),
            pl.BlockSpec((4 * EMBED_DIM, CPR), lambda j: (0, 0)),
            pl.BlockSpec((HIDDEN_DIM, 10), lambda j: (0, 0)),
            pl.BlockSpec((HIDDEN_DIM, 1), lambda j: (0, 0)),
            pl.BlockSpec((HIDDEN_DIM, 4 * EMBED_DIM), lambda j: (0, 0)),
            pl.BlockSpec((HIDDEN_DIM, HIDDEN_DIM), lambda j: (0, 0)),
            pl.BlockSpec((HIDDEN_DIM, 1), lambda j: (0, 0)),
            pl.BlockSpec((1, HIDDEN_DIM), lambda j: (0, 0)),
            pl.BlockSpec((1, 1), lambda j: (0, 0)),
        ],
        out_specs=pl.BlockSpec((1, NB_TC), lambda j: (0, j)),
        out_shape=jax.ShapeDtypeStruct((1, batch), jnp.float32),
        compiler_params=pltpu.CompilerParams(
            dimension_semantics=("parallel",)),
    )(toks_t, scalars_t, bd, w1t, b1c, w2at, w2bt, b2c, w3t, b3c)


def _head_body(cnt_ref, scal_ref, bd_ref, sel0t_ref, rept_ref, w1t_ref,
               b1_ref, w2at_ref, w2bt_ref, b2_ref, w3t_ref, b3_ref,
               out_ref):
    counts_t = cnt_ref[...]
    li = lax.broadcasted_iota(jnp.int32, (4, 1), 0)
    seg_lens = jnp.where(li == 2, 34.0, 11.0)
    z = jnp.dot(sel0t_ref[...], counts_t,
                preferred_element_type=jnp.float32)
    recip = 1.0 / jnp.maximum(seg_lens - z, 1.0)
    pooled_raw = jnp.dot(bd_ref[...], counts_t,
                         preferred_element_type=jnp.float32)
    scale = jnp.dot(rept_ref[...], recip,
                    preferred_element_type=jnp.float32)
    pooled = pooled_raw * scale
    sf = jnp.maximum(
        jnp.dot(w1t_ref[...], scal_ref[...],
                preferred_element_type=jnp.float32) + b1_ref[...], 0.0)
    h = jnp.maximum(
        jnp.dot(w2at_ref[...], pooled, preferred_element_type=jnp.float32)
        + jnp.dot(w2bt_ref[...], sf, preferred_element_type=jnp.float32)
        + b2_ref[...], 0.0)
    out_ref[...] = (jnp.dot(w3t_ref[...], h,
                            preferred_element_type=jnp.float32)
                    + b3_ref[...])


@jax.jit
def _head_call(cnt_t, scalars_t, bd, sel0t, rept, w1t, b1c, w2at, w2bt,
               b2c, w3t, b3c):
    batch = cnt_t.shape[1]
    grid = (batch // NB,)
    return pl.pallas_call(
        _head_body,
        grid=grid,
        in_specs=[
            pl.BlockSpec((CPR, NB), lambda j: (0, j)),
            pl.BlockSpec((10, NB), lambda j: (0, j)),
            pl.BlockSpec((4 * EMBED_DIM, CPR), lambda j: (0, 0)),
            pl.BlockSpec((4, CPR), lambda j: (0, 0)),
            pl.BlockSpec((4 * EMBED_DIM, 4), lambda j: (0, 0)),
            pl.BlockSpec((HIDDEN_DIM, 10), lambda j: (0, 0)),
            pl.BlockSpec((HIDDEN_DIM, 1), lambda j: (0, 0)),
            pl.BlockSpec((HIDDEN_DIM, 4 * EMBED_DIM), lambda j: (0, 0)),
            pl.BlockSpec((HIDDEN_DIM, HIDDEN_DIM), lambda j: (0, 0)),
            pl.BlockSpec((HIDDEN_DIM, 1), lambda j: (0, 0)),
            pl.BlockSpec((1, HIDDEN_DIM), lambda j: (0, 0)),
            pl.BlockSpec((1, 1), lambda j: (0, 0)),
        ],
        out_specs=pl.BlockSpec((1, NB), lambda j: (0, j)),
        out_shape=jax.ShapeDtypeStruct((1, batch), jnp.float32),
        compiler_params=pltpu.CompilerParams(
            dimension_semantics=("parallel",)),
    )(cnt_t, scalars_t, bd, sel0t, rept, w1t, b1c, w2at, w2bt, b2c, w3t,
      b3c)


SC_FRAC_NUM = 1
SC_FRAC_DEN = 4


def kernel(mut_tokens, wt_tokens, hla_tokens, delta_tokens, scalars,
           embedding, W1, b1, W2, b2, W3, b3):
    batch = mut_tokens.shape[0]
    sc_rows = (batch * SC_FRAC_NUM // SC_FRAC_DEN) // (NW * 16) * (NW * 16)
    rpw = sc_rows // NW
    toks_t = jnp.concatenate(
        [mut_tokens.T, wt_tokens.T, hla_tokens.T, delta_tokens.T],
        axis=0).astype(jnp.int32)

    cnt_t = _sc_counts(toks_t, sc_rows=sc_rows)

    ezt = embedding.at[0].set(0.0)
    ezt = jnp.pad(ezt, ((0, VPAD - VOCAB), (0, 0)))
    bdt = jnp.zeros((CPR, 4 * EMBED_DIM), jnp.float32)
    sel0 = jnp.zeros((CPR, 4), jnp.float32)
    rep = jnp.zeros((4, 4 * EMBED_DIM), jnp.float32)
    for s in range(4):
        bdt = bdt.at[s * VPAD:(s + 1) * VPAD,
                     s * EMBED_DIM:(s + 1) * EMBED_DIM].set(ezt)
        sel0 = sel0.at[s * VPAD, s].set(1.0)
        rep = rep.at[s, s * EMBED_DIM:(s + 1) * EMBED_DIM].set(1.0)

    scalars_t = scalars.T
    out_tc = _tc_call(toks_t, scalars_t, bdt.T,
                      W1.T, b1[:, None], W2[:64].T, W2[64:].T, b2[:, None],
                      W3.T, b3[:, None], col0=sc_rows)

    out_sc = _head_call(cnt_t, scalars_t[:, :sc_rows], bdt.T, sel0.T,
                        rep.T, W1.T, b1[:, None], W2[:64].T, W2[64:].T,
                        b2[:, None], W3.T, b3[:, None])
    return jnp.concatenate([out_sc[0], out_tc[0]])


# speedup vs baseline: 1.4051x; 1.1031x over previous
import functools

import jax
import jax.numpy as jnp
from jax import lax
from jax.experimental import pallas as pl
from jax.experimental.pallas import tpu as pltpu
from jax.experimental.pallas import tpu_sc as plsc

EMBED_DIM = 16
HIDDEN_DIM = 32
VOCAB = 21
VPAD = 24
CPR = 4 * VPAD
SEG_START = (0, 11, 22, 56)
SEG_LEN = (11, 11, 34, 11)
TOK_TOTAL = 67
NW = 32
NB = 4096


def _sc_body(toks_hbm, cnt_hbm, toks_v, cnt_v):
    cid = lax.axis_index("c")
    sid = lax.axis_index("s")
    wid = cid * 16 + sid
    rpw = toks_v.shape[1]
    pltpu.sync_copy(toks_hbm.at[:, pl.ds(wid * rpw, rpw)], toks_v)

    zeros16 = jnp.zeros((16,), jnp.float32)

    def zbody(i, carry):
        for slot in range(CPR):
            cnt_v[slot, pl.ds(i * 16, 16)] = zeros16
        return carry

    lax.fori_loop(0, rpw // 16, zbody, 0)

    laneiota = lax.iota(jnp.int32, 16)
    ones16 = jnp.ones((16,), jnp.float32)

    def gbody(g, carry):
        rows = laneiota + g * 16
        for s in range(4):
            for p in range(SEG_LEN[s]):
                tok = toks_v[SEG_START[s] + p, pl.ds(g * 16, 16)]
                plsc.addupdate_scatter(cnt_v, [tok + s * VPAD, rows],
                                       ones16)
        return carry

    lax.fori_loop(0, rpw // 16, gbody, 0)
    pltpu.sync_copy(cnt_v, cnt_hbm.at[:, pl.ds(wid * rpw, rpw)])


@functools.partial(jax.jit, static_argnames=("sc_rows",))
def _sc_counts(toks_t, sc_rows):
    rpw = sc_rows // NW
    mesh = plsc.VectorSubcoreMesh(core_axis_name="c", subcore_axis_name="s")
    f = pl.kernel(
        _sc_body,
        out_type=jax.ShapeDtypeStruct((CPR, sc_rows), jnp.float32),
        mesh=mesh,
        scratch_types=[
            pltpu.VMEM((TOK_TOTAL, rpw), jnp.int32),
            pltpu.VMEM((CPR, rpw), jnp.float32),
        ],
        compiler_params=pltpu.CompilerParams(needs_layout_passes=False),
    )
    return f(toks_t)


NB_TC = 4096


def _tc_body(toks_ref, scal_ref, bd_ref, w1t_ref, b1_ref, w2at_ref,
             w2bt_ref, b2_ref, w3t_ref, b3_ref, out_ref):
    toks = toks_ref[...]
    scal = scal_ref[...]
    viota = jax.lax.broadcasted_iota(jnp.int32, (VPAD, NB_TC), 0)
    slabs = []
    for seg_idx in range(4):
        start, length = SEG_START[seg_idx], SEG_LEN[seg_idx]
        parts = [jnp.zeros((VPAD, NB_TC), jnp.float32) for _ in range(2)]
        for p in range(length):
            tokp = toks[start + p, :][None, :]
            parts[p % 2] = parts[p % 2] + jnp.where(viota == tokp, 1.0, 0.0)
        slab = parts[0] + parts[1]
        denom = jnp.maximum(float(length) - slab[0:1, :], 1.0)
        slabs.append(slab * (1.0 / denom))
    counts = jnp.concatenate(slabs, axis=0)
    pooled = jnp.dot(bd_ref[...], counts,
                     preferred_element_type=jnp.float32)
    sf = jnp.maximum(
        jnp.dot(w1t_ref[...], scal,
                preferred_element_type=jnp.float32) + b1_ref[...], 0.0)
    h = jnp.maximum(
        jnp.dot(w2at_ref[...], pooled, preferred_element_type=jnp.float32)
        + jnp.dot(w2bt_ref[...], sf, preferred_element_type=jnp.float32)
        + b2_ref[...], 0.0)
    out_ref[...] = (jnp.dot(w3t_ref[...], h,
                            preferred_element_type=jnp.float32)
                    + b3_ref[...])


@functools.partial(jax.jit, static_argnames=("col0",))
def _tc_call(toks_t, scalars_t, bd, w1t, b1c, w2at, w2bt, b2c, w3t, b3c,
             col0=0):
    batch = toks_t.shape[1] - col0
    grid = (batch // NB_TC,)
    off = col0 // NB_TC
    return pl.pallas_call(
        _tc_body,
        grid=grid,
        in_specs=[
            pl.BlockSpec((TOK_TOTAL, NB_TC), lambda j: (0, j + off)),
            pl.BlockSpec((10, NB_TC), lambda j: (0, j + off)),
            pl.BlockSpec((4 * EMBED_DIM, CPR), lambda j: (0, 0)),
            pl.BlockSpec((HIDDEN_DIM, 10), lambda j: (0, 0)),
            pl.BlockSpec((HIDDEN_DIM, 1), lambda j: (0, 0)),
            pl.BlockSpec((HIDDEN_DIM, 4 * EMBED_DIM), lambda j: (0, 0)),
            pl.BlockSpec((HIDDEN_DIM, HIDDEN_DIM), lambda j: (0, 0)),
            pl.BlockSpec((HIDDEN_DIM, 1), lambda j: (0, 0)),
            pl.BlockSpec((1, HIDDEN_DIM), lambda j: (0, 0)),
            pl.BlockSpec((1, 1), lambda j: (0, 0)),
        ],
        out_specs=pl.BlockSpec((1, NB_TC), lambda j: (0, j)),
        out_shape=jax.ShapeDtypeStruct((1, batch), jnp.float32),
        compiler_params=pltpu.CompilerParams(
            dimension_semantics=("parallel",)),
    )(toks_t, scalars_t, bd, w1t, b1c, w2at, w2bt, b2c, w3t, b3c)


def _head_body(cnt_ref, scal_ref, bd_ref, sel0t_ref, rept_ref, w1t_ref,
               b1_ref, w2at_ref, w2bt_ref, b2_ref, w3t_ref, b3_ref,
               out_ref):
    counts_t = cnt_ref[...]
    li = lax.broadcasted_iota(jnp.int32, (4, 1), 0)
    seg_lens = jnp.where(li == 2, 34.0, 11.0)
    z = jnp.dot(sel0t_ref[...], counts_t,
                preferred_element_type=jnp.float32)
    recip = 1.0 / jnp.maximum(seg_lens - z, 1.0)
    pooled_raw = jnp.dot(bd_ref[...], counts_t,
                         preferred_element_type=jnp.float32)
    scale = jnp.dot(rept_ref[...], recip,
                    preferred_element_type=jnp.float32)
    pooled = pooled_raw * scale
    sf = jnp.maximum(
        jnp.dot(w1t_ref[...], scal_ref[...],
                preferred_element_type=jnp.float32) + b1_ref[...], 0.0)
    h = jnp.maximum(
        jnp.dot(w2at_ref[...], pooled, preferred_element_type=jnp.float32)
        + jnp.dot(w2bt_ref[...], sf, preferred_element_type=jnp.float32)
        + b2_ref[...], 0.0)
    out_ref[...] = (jnp.dot(w3t_ref[...], h,
                            preferred_element_type=jnp.float32)
                    + b3_ref[...])


@jax.jit
def _head_call(cnt_t, scalars_t, bd, sel0t, rept, w1t, b1c, w2at, w2bt,
               b2c, w3t, b3c):
    batch = cnt_t.shape[1]
    grid = (batch // NB,)
    return pl.pallas_call(
        _head_body,
        grid=grid,
        in_specs=[
            pl.BlockSpec((CPR, NB), lambda j: (0, j)),
            pl.BlockSpec((10, NB), lambda j: (0, j)),
            pl.BlockSpec((4 * EMBED_DIM, CPR), lambda j: (0, 0)),
            pl.BlockSpec((4, CPR), lambda j: (0, 0)),
            pl.BlockSpec((4 * EMBED_DIM, 4), lambda j: (0, 0)),
            pl.BlockSpec((HIDDEN_DIM, 10), lambda j: (0, 0)),
            pl.BlockSpec((HIDDEN_DIM, 1), lambda j: (0, 0)),
            pl.BlockSpec((HIDDEN_DIM, 4 * EMBED_DIM), lambda j: (0, 0)),
            pl.BlockSpec((HIDDEN_DIM, HIDDEN_DIM), lambda j: (0, 0)),
            pl.BlockSpec((HIDDEN_DIM, 1), lambda j: (0, 0)),
            pl.BlockSpec((1, HIDDEN_DIM), lambda j: (0, 0)),
            pl.BlockSpec((1, 1), lambda j: (0, 0)),
        ],
        out_specs=pl.BlockSpec((1, NB), lambda j: (0, j)),
        out_shape=jax.ShapeDtypeStruct((1, batch), jnp.float32),
        compiler_params=pltpu.CompilerParams(
            dimension_semantics=("parallel",)),
    )(cnt_t, scalars_t, bd, sel0t, rept, w1t, b1c, w2at, w2bt, b2c, w3t,
      b3c)


SC_FRAC_NUM = 1
SC_FRAC_DEN = 4


def _fused_body(toks_ref, cnt_ref, scal_ref, bd_ref, sel0t_ref, rept_ref,
                w1t_ref, b1_ref, w2at_ref, w2bt_ref, b2_ref, w3t_ref,
                b3_ref, out_ref):
    j = pl.program_id(0)
    scal = scal_ref[...]
    sf = jnp.maximum(
        jnp.dot(w1t_ref[...], scal,
                preferred_element_type=jnp.float32) + b1_ref[...], 0.0)

    def finish(pooled):
        h = jnp.maximum(
            jnp.dot(w2at_ref[...], pooled,
                    preferred_element_type=jnp.float32)
            + jnp.dot(w2bt_ref[...], sf,
                      preferred_element_type=jnp.float32)
            + b2_ref[...], 0.0)
        out_ref[...] = (jnp.dot(w3t_ref[...], h,
                                preferred_element_type=jnp.float32)
                        + b3_ref[...])

    @pl.when(j == 0)
    def _head_mode():
        counts_t = cnt_ref[...]
        li = lax.broadcasted_iota(jnp.int32, (4, 1), 0)
        seg_lens = jnp.where(li == 2, 34.0, 11.0)
        z = jnp.dot(sel0t_ref[...], counts_t,
                    preferred_element_type=jnp.float32)
        recip = 1.0 / jnp.maximum(seg_lens - z, 1.0)
        pooled_raw = jnp.dot(bd_ref[...], counts_t,
                             preferred_element_type=jnp.float32)
        scale = jnp.dot(rept_ref[...], recip,
                        preferred_element_type=jnp.float32)
        finish(pooled_raw * scale)

    @pl.when(j > 0)
    def _direct_mode():
        toks = toks_ref[...]
        viota = jax.lax.broadcasted_iota(jnp.int32, (VPAD, NB_TC), 0)
        slabs = []
        for seg_idx in range(4):
            start, length = SEG_START[seg_idx], SEG_LEN[seg_idx]
            parts = [jnp.zeros((VPAD, NB_TC), jnp.float32)
                     for _ in range(2)]
            for pp in range(length):
                tokp = toks[start + pp, :][None, :]
                parts[pp % 2] = parts[pp % 2] + jnp.where(
                    viota == tokp, 1.0, 0.0)
            slab = parts[0] + parts[1]
            denom = jnp.maximum(float(length) - slab[0:1, :], 1.0)
            slabs.append(slab * (1.0 / denom))
        counts = jnp.concatenate(slabs, axis=0)
        pooled = jnp.dot(bd_ref[...], counts,
                         preferred_element_type=jnp.float32)
        finish(pooled)


@jax.jit
def _fused_call(toks_t, cnt_t, scalars_t, bd, sel0t, rept, w1t, b1c, w2at,
                w2bt, b2c, w3t, b3c):
    batch = toks_t.shape[1]
    grid = (batch // NB_TC,)
    return pl.pallas_call(
        _fused_body,
        grid=grid,
        in_specs=[
            pl.BlockSpec((TOK_TOTAL, NB_TC), lambda j: (0, j)),
            pl.BlockSpec((CPR, NB_TC), lambda j: (0, 0)),
            pl.BlockSpec((10, NB_TC), lambda j: (0, j)),
            pl.BlockSpec((4 * EMBED_DIM, CPR), lambda j: (0, 0)),
            pl.BlockSpec((4, CPR), lambda j: (0, 0)),
            pl.BlockSpec((4 * EMBED_DIM, 4), lambda j: (0, 0)),
            pl.BlockSpec((HIDDEN_DIM, 10), lambda j: (0, 0)),
            pl.BlockSpec((HIDDEN_DIM, 1), lambda j: (0, 0)),
            pl.BlockSpec((HIDDEN_DIM, 4 * EMBED_DIM), lambda j: (0, 0)),
            pl.BlockSpec((HIDDEN_DIM, HIDDEN_DIM), lambda j: (0, 0)),
            pl.BlockSpec((HIDDEN_DIM, 1), lambda j: (0, 0)),
            pl.BlockSpec((1, HIDDEN_DIM), lambda j: (0, 0)),
            pl.BlockSpec((1, 1), lambda j: (0, 0)),
        ],
        out_specs=pl.BlockSpec((1, NB_TC), lambda j: (0, j)),
        out_shape=jax.ShapeDtypeStruct((1, batch), jnp.float32),
        compiler_params=pltpu.CompilerParams(
            dimension_semantics=("arbitrary",)),
    )(toks_t, cnt_t, scalars_t, bd, sel0t, rept, w1t, b1c, w2at, w2bt,
      b2c, w3t, b3c)


def kernel(mut_tokens, wt_tokens, hla_tokens, delta_tokens, scalars,
           embedding, W1, b1, W2, b2, W3, b3):
    batch = mut_tokens.shape[0]
    sc_rows = (batch * SC_FRAC_NUM // SC_FRAC_DEN) // (NW * 16) * (NW * 16)
    toks_t = jnp.concatenate(
        [mut_tokens.T, wt_tokens.T, hla_tokens.T, delta_tokens.T],
        axis=0).astype(jnp.int32)

    cnt_t = _sc_counts(toks_t, sc_rows=sc_rows)

    ezt = embedding.at[0].set(0.0)
    ezt = jnp.pad(ezt, ((0, VPAD - VOCAB), (0, 0)))
    bdt = jnp.zeros((CPR, 4 * EMBED_DIM), jnp.float32)
    sel0 = jnp.zeros((CPR, 4), jnp.float32)
    rep = jnp.zeros((4, 4 * EMBED_DIM), jnp.float32)
    for s in range(4):
        bdt = bdt.at[s * VPAD:(s + 1) * VPAD,
                     s * EMBED_DIM:(s + 1) * EMBED_DIM].set(ezt)
        sel0 = sel0.at[s * VPAD, s].set(1.0)
        rep = rep.at[s, s * EMBED_DIM:(s + 1) * EMBED_DIM].set(1.0)

    out = _fused_call(toks_t, cnt_t, scalars.T, bdt.T, sel0.T, rep.T,
                      W1.T, b1[:, None], W2[:64].T, W2[64:].T,
                      b2[:, None], W3.T, b3[:, None])
    return out[0]


# speedup vs baseline: 1.4098x; 1.0034x over previous
import functools

import jax
import jax.numpy as jnp
from jax import lax
from jax.experimental import pallas as pl
from jax.experimental.pallas import tpu as pltpu
from jax.experimental.pallas import tpu_sc as plsc

EMBED_DIM = 16
HIDDEN_DIM = 32
VOCAB = 21
VPAD = 24
CPR = 4 * VPAD
SEG_START = (0, 11, 22, 56)
SEG_LEN = (11, 11, 34, 11)
TOK_TOTAL = 67
NW = 32
NB = 4096


def _sc_body(toks_hbm, cnt_hbm, toks_v, cnt_v):
    cid = lax.axis_index("c")
    sid = lax.axis_index("s")
    wid = cid * 16 + sid
    rpw = toks_v.shape[1]
    pltpu.sync_copy(toks_hbm.at[:, pl.ds(wid * rpw, rpw)], toks_v)

    zeros16 = jnp.zeros((16,), jnp.float32)

    def zbody(i, carry):
        for slot in range(CPR):
            cnt_v[slot, pl.ds(i * 16, 16)] = zeros16
        return carry

    lax.fori_loop(0, rpw // 16, zbody, 0)

    laneiota = lax.iota(jnp.int32, 16)
    ones16 = jnp.ones((16,), jnp.float32)

    def gbody(g, carry):
        rows = laneiota + g * 16
        for s in range(4):
            for p in range(SEG_LEN[s]):
                tok = toks_v[SEG_START[s] + p, pl.ds(g * 16, 16)]
                plsc.addupdate_scatter(cnt_v, [tok + s * VPAD, rows],
                                       ones16)
        return carry

    lax.fori_loop(0, rpw // 16, gbody, 0)
    pltpu.sync_copy(cnt_v, cnt_hbm.at[:, pl.ds(wid * rpw, rpw)])


@functools.partial(jax.jit, static_argnames=("sc_rows",))
def _sc_counts(toks_t, sc_rows):
    rpw = sc_rows // NW
    mesh = plsc.VectorSubcoreMesh(core_axis_name="c", subcore_axis_name="s")
    f = pl.kernel(
        _sc_body,
        out_type=jax.ShapeDtypeStruct((CPR, sc_rows), jnp.float32),
        mesh=mesh,
        scratch_types=[
            pltpu.VMEM((TOK_TOTAL, rpw), jnp.int32),
            pltpu.VMEM((CPR, rpw), jnp.float32),
        ],
        compiler_params=pltpu.CompilerParams(needs_layout_passes=False),
    )
    return f(toks_t)


NB_TC = 4096


def _tc_body(toks_ref, scal_ref, bd_ref, w1t_ref, b1_ref, w2at_ref,
             w2bt_ref, b2_ref, w3t_ref, b3_ref, out_ref):
    toks = toks_ref[...]
    scal = scal_ref[...]
    viota = jax.lax.broadcasted_iota(jnp.int32, (VPAD, NB_TC), 0)
    slabs = []
    for seg_idx in range(4):
        start, length = SEG_START[seg_idx], SEG_LEN[seg_idx]
        parts = [jnp.zeros((VPAD, NB_TC), jnp.float32) for _ in range(2)]
        for p in range(length):
            tokp = toks[start + p, :][None, :]
            parts[p % 2] = parts[p % 2] + jnp.where(viota == tokp, 1.0, 0.0)
        slab = parts[0] + parts[1]
        denom = jnp.maximum(float(length) - slab[0:1, :], 1.0)
        slabs.append(slab * (1.0 / denom))
    counts = jnp.concatenate(slabs, axis=0)
    pooled = jnp.dot(bd_ref[...], counts,
                     preferred_element_type=jnp.float32)
    sf = jnp.maximum(
        jnp.dot(w1t_ref[...], scal,
                preferred_element_type=jnp.float32) + b1_ref[...], 0.0)
    h = jnp.maximum(
        jnp.dot(w2at_ref[...], pooled, preferred_element_type=jnp.float32)
        + jnp.dot(w2bt_ref[...], sf, preferred_element_type=jnp.float32)
        + b2_ref[...], 0.0)
    out_ref[...] = (jnp.dot(w3t_ref[...], h,
                            preferred_element_type=jnp.float32)
                    + b3_ref[...])


@functools.partial(jax.jit, static_argnames=("col0",))
def _tc_call(toks_t, scalars_t, bd, w1t, b1c, w2at, w2bt, b2c, w3t, b3c,
             col0=0):
    batch = toks_t.shape[1] - col0
    grid = (batch // NB_TC,)
    off = col0 // NB_TC
    return pl.pallas_call(
        _tc_body,
        grid=grid,
        in_specs=[
            pl.BlockSpec((TOK_TOTAL, NB_TC), lambda j: (0, j + off)),
            pl.BlockSpec((10, NB_TC), lambda j: (0, j + off)),
            pl.BlockSpec((4 * EMBED_DIM, CPR), lambda j: (0, 0)),
            pl.BlockSpec((HIDDEN_DIM, 10), lambda j: (0, 0)),
            pl.BlockSpec((HIDDEN_DIM, 1), lambda j: (0, 0)),
            pl.BlockSpec((HIDDEN_DIM, 4 * EMBED_DIM), lambda j: (0, 0)),
            pl.BlockSpec((HIDDEN_DIM, HIDDEN_DIM), lambda j: (0, 0)),
            pl.BlockSpec((HIDDEN_DIM, 1), lambda j: (0, 0)),
            pl.BlockSpec((1, HIDDEN_DIM), lambda j: (0, 0)),
            pl.BlockSpec((1, 1), lambda j: (0, 0)),
        ],
        out_specs=pl.BlockSpec((1, NB_TC), lambda j: (0, j)),
        out_shape=jax.ShapeDtypeStruct((1, batch), jnp.float32),
        compiler_params=pltpu.CompilerParams(
            dimension_semantics=("parallel",)),
    )(toks_t, scalars_t, bd, w1t, b1c, w2at, w2bt, b2c, w3t, b3c)


def _head_body(cnt_ref, scal_ref, bd_ref, sel0t_ref, rept_ref, w1t_ref,
               b1_ref, w2at_ref, w2bt_ref, b2_ref, w3t_ref, b3_ref,
               out_ref):
    counts_t = cnt_ref[...]
    li = lax.broadcasted_iota(jnp.int32, (4, 1), 0)
    seg_lens = jnp.where(li == 2, 34.0, 11.0)
    z = jnp.dot(sel0t_ref[...], counts_t,
                preferred_element_type=jnp.float32)
    recip = 1.0 / jnp.maximum(seg_lens - z, 1.0)
    pooled_raw = jnp.dot(bd_ref[...], counts_t,
                         preferred_element_type=jnp.float32)
    scale = jnp.dot(rept_ref[...], recip,
                    preferred_element_type=jnp.float32)
    pooled = pooled_raw * scale
    sf = jnp.maximum(
        jnp.dot(w1t_ref[...], scal_ref[...],
                preferred_element_type=jnp.float32) + b1_ref[...], 0.0)
    h = jnp.maximum(
        jnp.dot(w2at_ref[...], pooled, preferred_element_type=jnp.float32)
        + jnp.dot(w2bt_ref[...], sf, preferred_element_type=jnp.float32)
        + b2_ref[...], 0.0)
    out_ref[...] = (jnp.dot(w3t_ref[...], h,
                            preferred_element_type=jnp.float32)
                    + b3_ref[...])


@jax.jit
def _head_call(cnt_t, scalars_t, bd, sel0t, rept, w1t, b1c, w2at, w2bt,
               b2c, w3t, b3c):
    batch = cnt_t.shape[1]
    grid = (batch // NB,)
    return pl.pallas_call(
        _head_body,
        grid=grid,
        in_specs=[
            pl.BlockSpec((CPR, NB), lambda j: (0, j)),
            pl.BlockSpec((10, NB), lambda j: (0, j)),
            pl.BlockSpec((4 * EMBED_DIM, CPR), lambda j: (0, 0)),
            pl.BlockSpec((4, CPR), lambda j: (0, 0)),
            pl.BlockSpec((4 * EMBED_DIM, 4), lambda j: (0, 0)),
            pl.BlockSpec((HIDDEN_DIM, 10), lambda j: (0, 0)),
            pl.BlockSpec((HIDDEN_DIM, 1), lambda j: (0, 0)),
            pl.BlockSpec((HIDDEN_DIM, 4 * EMBED_DIM), lambda j: (0, 0)),
            pl.BlockSpec((HIDDEN_DIM, HIDDEN_DIM), lambda j: (0, 0)),
            pl.BlockSpec((HIDDEN_DIM, 1), lambda j: (0, 0)),
            pl.BlockSpec((1, HIDDEN_DIM), lambda j: (0, 0)),
            pl.BlockSpec((1, 1), lambda j: (0, 0)),
        ],
        out_specs=pl.BlockSpec((1, NB), lambda j: (0, j)),
        out_shape=jax.ShapeDtypeStruct((1, batch), jnp.float32),
        compiler_params=pltpu.CompilerParams(
            dimension_semantics=("parallel",)),
    )(cnt_t, scalars_t, bd, sel0t, rept, w1t, b1c, w2at, w2bt, b2c, w3t,
      b3c)


SC_FRAC_NUM = 1
SC_FRAC_DEN = 4


def _fused_body(toks_ref, cnt_ref, scal_ref, bd_ref, sel0t_ref, rept_ref,
                w1t_ref, b1_ref, w2at_ref, w2bt_ref, b2_ref, w3t_ref,
                b3_ref, out_ref):
    j = pl.program_id(0)
    scal = scal_ref[...]
    sf = jnp.maximum(
        jnp.dot(w1t_ref[...], scal,
                preferred_element_type=jnp.float32) + b1_ref[...], 0.0)

    def finish(pooled):
        h = jnp.maximum(
            jnp.dot(w2at_ref[...], pooled,
                    preferred_element_type=jnp.float32)
            + jnp.dot(w2bt_ref[...], sf,
                      preferred_element_type=jnp.float32)
            + b2_ref[...], 0.0)
        out_ref[...] = (jnp.dot(w3t_ref[...], h,
                                preferred_element_type=jnp.float32)
                        + b3_ref[...])

    @pl.when(j == 0)
    def _head_mode():
        counts_t = cnt_ref[...]
        li = lax.broadcasted_iota(jnp.int32, (4, 1), 0)
        seg_lens = jnp.where(li == 2, 34.0, 11.0)
        z = jnp.dot(sel0t_ref[...], counts_t,
                    preferred_element_type=jnp.float32)
        recip = 1.0 / jnp.maximum(seg_lens - z, 1.0)
        pooled_raw = jnp.dot(bd_ref[...], counts_t,
                             preferred_element_type=jnp.float32)
        scale = jnp.dot(rept_ref[...], recip,
                        preferred_element_type=jnp.float32)
        finish(pooled_raw * scale)

    @pl.when(j > 0)
    def _direct_mode():
        toks = toks_ref[...]
        viota = jax.lax.broadcasted_iota(jnp.int32, (VPAD, NB_TC), 0)
        slabs = []
        for seg_idx in range(4):
            start, length = SEG_START[seg_idx], SEG_LEN[seg_idx]
            parts = [jnp.zeros((VPAD, NB_TC), jnp.float32)
                     for _ in range(2)]
            for pp in range(length):
                tokp = toks[start + pp, :][None, :]
                parts[pp % 2] = parts[pp % 2] + jnp.where(
                    viota == tokp, 1.0, 0.0)
            slab = parts[0] + parts[1]
            denom = jnp.maximum(float(length) - slab[0:1, :], 1.0)
            slabs.append(slab * (1.0 / denom))
        counts = jnp.concatenate(slabs, axis=0)
        pooled = jnp.dot(bd_ref[...], counts,
                         preferred_element_type=jnp.float32)
        finish(pooled)


@jax.jit
def _fused_call(toks_t, cnt_t, scalars_t, bd, sel0t, rept, w1t, b1c, w2at,
                w2bt, b2c, w3t, b3c):
    batch = toks_t.shape[1]
    grid = (batch // NB_TC,)
    return pl.pallas_call(
        _fused_body,
        grid=grid,
        in_specs=[
            pl.BlockSpec((TOK_TOTAL, NB_TC), lambda j: (0, j)),
            pl.BlockSpec((CPR, NB_TC), lambda j: (0, 0)),
            pl.BlockSpec((10, NB_TC), lambda j: (0, j)),
            pl.BlockSpec((4 * EMBED_DIM, CPR), lambda j: (0, 0)),
            pl.BlockSpec((4, CPR), lambda j: (0, 0)),
            pl.BlockSpec((4 * EMBED_DIM, 4), lambda j: (0, 0)),
            pl.BlockSpec((HIDDEN_DIM, 10), lambda j: (0, 0)),
            pl.BlockSpec((HIDDEN_DIM, 1), lambda j: (0, 0)),
            pl.BlockSpec((HIDDEN_DIM, 4 * EMBED_DIM), lambda j: (0, 0)),
            pl.BlockSpec((HIDDEN_DIM, HIDDEN_DIM), lambda j: (0, 0)),
            pl.BlockSpec((HIDDEN_DIM, 1), lambda j: (0, 0)),
            pl.BlockSpec((1, HIDDEN_DIM), lambda j: (0, 0)),
            pl.BlockSpec((1, 1), lambda j: (0, 0)),
        ],
        out_specs=pl.BlockSpec((1, NB_TC), lambda j: (0, j)),
        out_shape=jax.ShapeDtypeStruct((1, batch), jnp.float32),
        compiler_params=pltpu.CompilerParams(
            dimension_semantics=("parallel",)),
    )(toks_t, cnt_t, scalars_t, bd, sel0t, rept, w1t, b1c, w2at, w2bt,
      b2c, w3t, b3c)


def kernel(mut_tokens, wt_tokens, hla_tokens, delta_tokens, scalars,
           embedding, W1, b1, W2, b2, W3, b3):
    batch = mut_tokens.shape[0]
    sc_rows = (batch * SC_FRAC_NUM // SC_FRAC_DEN) // (NW * 16) * (NW * 16)
    toks_t = jnp.concatenate(
        [mut_tokens.T, wt_tokens.T, hla_tokens.T, delta_tokens.T],
        axis=0).astype(jnp.int32)

    cnt_t = _sc_counts(toks_t, sc_rows=sc_rows)

    ezt = embedding.at[0].set(0.0)
    ezt = jnp.pad(ezt, ((0, VPAD - VOCAB), (0, 0)))
    bdt = jnp.zeros((CPR, 4 * EMBED_DIM), jnp.float32)
    sel0 = jnp.zeros((CPR, 4), jnp.float32)
    rep = jnp.zeros((4, 4 * EMBED_DIM), jnp.float32)
    for s in range(4):
        bdt = bdt.at[s * VPAD:(s + 1) * VPAD,
                     s * EMBED_DIM:(s + 1) * EMBED_DIM].set(ezt)
        sel0 = sel0.at[s * VPAD, s].set(1.0)
        rep = rep.at[s, s * EMBED_DIM:(s + 1) * EMBED_DIM].set(1.0)

    out = _fused_call(toks_t, cnt_t, scalars.T, bdt.T, sel0.T, rep.T,
                      W1.T, b1[:, None], W2[:64].T, W2[64:].T,
                      b2[:, None], W3.T, b3[:, None])
    return out[0]


# speedup vs baseline: 1.4107x; 1.0006x over previous
import functools

import jax
import jax.numpy as jnp
from jax import lax
from jax.experimental import pallas as pl
from jax.experimental.pallas import tpu as pltpu
from jax.experimental.pallas import tpu_sc as plsc

EMBED_DIM = 16
HIDDEN_DIM = 32
VOCAB = 21
VPAD = 24
CPR = 4 * VPAD
SEG_START = (0, 11, 22, 56)
SEG_LEN = (11, 11, 34, 11)
TOK_TOTAL = 67
NW = 32
NB = 4096


def _sc_body(toks_hbm, cnt_hbm, toks_v, cnt_v):
    cid = lax.axis_index("c")
    sid = lax.axis_index("s")
    wid = cid * 16 + sid
    rpw = toks_v.shape[1]
    pltpu.sync_copy(toks_hbm.at[:, pl.ds(wid * rpw, rpw)], toks_v)

    zeros16 = jnp.zeros((16,), jnp.float32)

    def zbody(i, carry):
        for slot in range(CPR):
            cnt_v[slot, pl.ds(i * 16, 16)] = zeros16
        return carry

    lax.fori_loop(0, rpw // 16, zbody, 0)

    laneiota = lax.iota(jnp.int32, 16)
    ones16 = jnp.ones((16,), jnp.float32)

    def gbody(g, carry):
        rows = laneiota + g * 16
        for s in range(4):
            for p in range(SEG_LEN[s]):
                tok = toks_v[SEG_START[s] + p, pl.ds(g * 16, 16)]
                plsc.addupdate_scatter(cnt_v, [tok + s * VPAD, rows],
                                       ones16)
        return carry

    lax.fori_loop(0, rpw // 16, gbody, 0)
    pltpu.sync_copy(cnt_v, cnt_hbm.at[:, pl.ds(wid * rpw, rpw)])


@functools.partial(jax.jit, static_argnames=("sc_rows",))
def _sc_counts(toks_t, sc_rows):
    rpw = sc_rows // NW
    mesh = plsc.VectorSubcoreMesh(core_axis_name="c", subcore_axis_name="s")
    f = pl.kernel(
        _sc_body,
        out_type=jax.ShapeDtypeStruct((CPR, sc_rows), jnp.float32),
        mesh=mesh,
        scratch_types=[
            pltpu.VMEM((TOK_TOTAL, rpw), jnp.int32),
            pltpu.VMEM((CPR, rpw), jnp.float32),
        ],
        compiler_params=pltpu.CompilerParams(needs_layout_passes=False),
    )
    return f(toks_t)


NB_TC = 4096


SC_FRAC_NUM = 1
SC_FRAC_DEN = 4


def _fused_body(toks_ref, cnt_ref, scal_ref, bd_ref, sel0t_ref, rept_ref,
                w1t_ref, b1_ref, w2at_ref, w2bt_ref, b2_ref, w3t_ref,
                b3_ref, out_ref):
    j = pl.program_id(0)
    scal = scal_ref[...]
    sf = jnp.maximum(
        jnp.dot(w1t_ref[...], scal,
                preferred_element_type=jnp.float32) + b1_ref[...], 0.0)

    def finish(pooled):
        h = jnp.maximum(
            jnp.dot(w2at_ref[...], pooled,
                    preferred_element_type=jnp.float32)
            + jnp.dot(w2bt_ref[...], sf,
                      preferred_element_type=jnp.float32)
            + b2_ref[...], 0.0)
        out_ref[...] = (jnp.dot(w3t_ref[...], h,
                                preferred_element_type=jnp.float32)
                        + b3_ref[...])

    @pl.when(j == 0)
    def _head_mode():
        counts_t = cnt_ref[...]
        li = lax.broadcasted_iota(jnp.int32, (4, 1), 0)
        seg_lens = jnp.where(li == 2, 34.0, 11.0)
        z = jnp.dot(sel0t_ref[...], counts_t,
                    preferred_element_type=jnp.float32)
        recip = 1.0 / jnp.maximum(seg_lens - z, 1.0)
        pooled_raw = jnp.dot(bd_ref[...], counts_t,
                             preferred_element_type=jnp.float32)
        scale = jnp.dot(rept_ref[...], recip,
                        preferred_element_type=jnp.float32)
        finish(pooled_raw * scale)

    @pl.when(j > 0)
    def _direct_mode():
        toks = toks_ref[...]
        viota = jax.lax.broadcasted_iota(jnp.int32, (VPAD, NB_TC), 0)
        slabs = []
        for seg_idx in range(4):
            start, length = SEG_START[seg_idx], SEG_LEN[seg_idx]
            parts = [jnp.zeros((VPAD, NB_TC), jnp.float32)
                     for _ in range(2)]
            for pp in range(length):
                tokp = toks[start + pp, :][None, :]
                parts[pp % 2] = parts[pp % 2] + jnp.where(
                    viota == tokp, 1.0, 0.0)
            slab = parts[0] + parts[1]
            denom = jnp.maximum(float(length) - slab[0:1, :], 1.0)
            slabs.append(slab * (1.0 / denom))
        counts = jnp.concatenate(slabs, axis=0)
        pooled = jnp.dot(bd_ref[...], counts,
                         preferred_element_type=jnp.float32)
        finish(pooled)


@jax.jit
def _fused_call(toks_t, cnt_t, scalars_t, bd, sel0t, rept, w1t, b1c, w2at,
                w2bt, b2c, w3t, b3c):
    batch = toks_t.shape[1]
    grid = (batch // NB_TC,)
    return pl.pallas_call(
        _fused_body,
        grid=grid,
        in_specs=[
            pl.BlockSpec((TOK_TOTAL, NB_TC), lambda j: (0, j)),
            pl.BlockSpec((CPR, NB_TC), lambda j: (0, 0)),
            pl.BlockSpec((10, NB_TC), lambda j: (0, j)),
            pl.BlockSpec((4 * EMBED_DIM, CPR), lambda j: (0, 0)),
            pl.BlockSpec((4, CPR), lambda j: (0, 0)),
            pl.BlockSpec((4 * EMBED_DIM, 4), lambda j: (0, 0)),
            pl.BlockSpec((HIDDEN_DIM, 10), lambda j: (0, 0)),
            pl.BlockSpec((HIDDEN_DIM, 1), lambda j: (0, 0)),
            pl.BlockSpec((HIDDEN_DIM, 4 * EMBED_DIM), lambda j: (0, 0)),
            pl.BlockSpec((HIDDEN_DIM, HIDDEN_DIM), lambda j: (0, 0)),
            pl.BlockSpec((HIDDEN_DIM, 1), lambda j: (0, 0)),
            pl.BlockSpec((1, HIDDEN_DIM), lambda j: (0, 0)),
            pl.BlockSpec((1, 1), lambda j: (0, 0)),
        ],
        out_specs=pl.BlockSpec((1, NB_TC), lambda j: (0, j)),
        out_shape=jax.ShapeDtypeStruct((1, batch), jnp.float32),
        compiler_params=pltpu.CompilerParams(
            dimension_semantics=("parallel",)),
    )(toks_t, cnt_t, scalars_t, bd, sel0t, rept, w1t, b1c, w2at, w2bt,
      b2c, w3t, b3c)


def kernel(mut_tokens, wt_tokens, hla_tokens, delta_tokens, scalars,
           embedding, W1, b1, W2, b2, W3, b3):
    batch = mut_tokens.shape[0]
    sc_rows = (batch * SC_FRAC_NUM // SC_FRAC_DEN) // (NW * 16) * (NW * 16)
    toks_t = jnp.concatenate(
        [mut_tokens.T, wt_tokens.T, hla_tokens.T, delta_tokens.T],
        axis=0).astype(jnp.int32)

    cnt_t = _sc_counts(toks_t, sc_rows=sc_rows)

    ezt = embedding.at[0].set(0.0)
    ezt = jnp.pad(ezt, ((0, VPAD - VOCAB), (0, 0)))
    bdt = jnp.zeros((CPR, 4 * EMBED_DIM), jnp.float32)
    sel0 = jnp.zeros((CPR, 4), jnp.float32)
    rep = jnp.zeros((4, 4 * EMBED_DIM), jnp.float32)
    for s in range(4):
        bdt = bdt.at[s * VPAD:(s + 1) * VPAD,
                     s * EMBED_DIM:(s + 1) * EMBED_DIM].set(ezt)
        sel0 = sel0.at[s * VPAD, s].set(1.0)
        rep = rep.at[s, s * EMBED_DIM:(s + 1) * EMBED_DIM].set(1.0)

    out = _fused_call(toks_t, cnt_t, scalars.T, bdt.T, sel0.T, rep.T,
                      W1.T, b1[:, None], W2[:64].T, W2[64:].T,
                      b2[:, None], W3.T, b3[:, None])
    return out[0]
